# Initial kernel scaffold; baseline (speedup 1.0000x reference)
import jax
import jax.numpy as jnp
from jax.experimental import pallas as pl


def kernel(tokens, vocab_map, embed_table):
    raise NotImplementedError("write your pallas kernel here")


# speedup vs baseline: 11.9035x; 11.9035x over previous
import functools

import jax
import jax.numpy as jnp
from jax import lax
from jax.experimental import pallas as pl
from jax.experimental.pallas import tpu as pltpu
from jax.experimental.pallas import tpu_sc as plsc

_VOCAB = 100000
_D = 64
_B = 4096
_S = 256

_NC = 2
_NS = 16
_NW = _NC * _NS

_TOTAL = _B * _S
_PER_W = _TOTAL // _NW
_C = 128
_NCHUNK = _PER_W // _C


def _make_kernel():
    mesh = plsc.VectorSubcoreMesh(core_axis_name="c", subcore_axis_name="s")

    @functools.partial(
        pl.kernel,
        mesh=mesh,
        out_type=jax.ShapeDtypeStruct((_TOTAL, _D), jnp.float32),
        scratch_types=[
            pltpu.VMEM((_C,), jnp.int32),
            pltpu.VMEM((_C,), jnp.int32),
            pltpu.VMEM((_C, _D), jnp.float32),
            pltpu.SemaphoreType.DMA,
        ],
        compiler_params=pltpu.CompilerParams(use_tc_tiling_on_sc=False),
    )
    def k(tok_hbm, vmap_hbm, table_hbm, out_hbm, tok_v, ids_v, emb_v, sem):
        wid = lax.axis_index("s") * _NC + lax.axis_index("c")
        base = wid * _PER_W

        def body(g, carry):
            off = base + g * _C
            pltpu.sync_copy(tok_hbm.at[pl.ds(off, _C)], tok_v)
            pltpu.async_copy(vmap_hbm.at[tok_v], ids_v, sem).wait()
            pltpu.async_copy(table_hbm.at[ids_v], emb_v, sem).wait()
            pltpu.sync_copy(emb_v, out_hbm.at[pl.ds(off, _C)])
            return carry

        lax.fori_loop(0, _NCHUNK, body, 0, unroll=False)

    return k


_k = _make_kernel()


def kernel(tokens, vocab_map, embed_table):
    b = tokens.shape[0]
    vmap_ext = jnp.concatenate(
        [vocab_map, jnp.array([0, 2], dtype=vocab_map.dtype)])
    bos = jnp.full((b, 1), _VOCAB, dtype=tokens.dtype)
    eos = jnp.full((b, 1), _VOCAB + 1, dtype=tokens.dtype)
    tok_ext = jnp.concatenate([bos, tokens, eos], axis=1).reshape(-1)
    out = _k(tok_ext, vmap_ext, embed_table)
    return out.reshape(b, _S, _D)


# speedup vs baseline: 16.3963x; 1.3774x over previous
import functools

import jax
import jax.numpy as jnp
from jax import lax
from jax.experimental import pallas as pl
from jax.experimental.pallas import tpu as pltpu
from jax.experimental.pallas import tpu_sc as plsc

_VOCAB = 100000
_D = 64
_B = 4096
_S = 256

_NC = 2
_NS = 16
_NW = _NC * _NS

_TOTAL = _B * _S
_PER_W = _TOTAL // _NW
_C = 128
_NCHUNK = _PER_W // _C
_NB = 4


def _make_kernel():
    mesh = plsc.VectorSubcoreMesh(core_axis_name="c", subcore_axis_name="s")

    @functools.partial(
        pl.kernel,
        mesh=mesh,
        out_type=jax.ShapeDtypeStruct((_TOTAL, _D), jnp.float32),
        scratch_types=[
            pltpu.VMEM((_NCHUNK, _C), jnp.int32),
            pltpu.VMEM((_NCHUNK, _C), jnp.int32),
            pltpu.VMEM((_NB, _C, _D), jnp.float32),
            pltpu.SemaphoreType.DMA((_NB,)),
            pltpu.SemaphoreType.DMA((_NB,)),
            pltpu.SemaphoreType.DMA((_NB,)),
        ],
        compiler_params=pltpu.CompilerParams(use_tc_tiling_on_sc=False),
    )
    def k(tok_hbm, vmap_hbm, table_hbm, out_hbm,
          tok_all, ids_all, emb, isem, gsem, ssem):
        wid = lax.axis_index("s") * _NC + lax.axis_index("c")
        base = wid * _PER_W
        brow = wid * _NCHUNK

        pltpu.sync_copy(tok_hbm.at[pl.ds(brow, _NCHUNK)], tok_all)

        def fire_ids(g, slot):
            pltpu.async_copy(vmap_hbm.at[tok_all.at[g]], ids_all.at[g],
                             isem.at[slot])

        def wait_ids(slot):
            pltpu.make_async_copy(vmap_hbm.at[pl.ds(0, _C)], ids_all.at[0],
                                  isem.at[slot]).wait()

        def fire_emb(g, slot):
            pltpu.async_copy(table_hbm.at[ids_all.at[g]], emb.at[slot],
                             gsem.at[slot])

        def wait_emb(slot):
            pltpu.make_async_copy(table_hbm.at[pl.ds(0, _C)], emb.at[slot],
                                  gsem.at[slot]).wait()

        def fire_out(g, slot):
            pltpu.async_copy(emb.at[slot], out_hbm.at[pl.ds(base + g * _C, _C)],
                             ssem.at[slot])

        def wait_out(slot):
            pltpu.make_async_copy(table_hbm.at[pl.ds(0, _C)], emb.at[slot],
                                  ssem.at[slot]).wait()

        def body(g, carry, *, ring_wait, ids_ahead, do_prev):
            r = lax.rem(g, _NB)
            wait_ids(r)
            if ring_wait:
                wait_out(r)
            fire_emb(g, r)
            if ids_ahead:
                fire_ids(g + _NB, r)
            if do_prev:
                r1 = lax.rem(g - 1 + _NB, _NB)
                wait_emb(r1)
                fire_out(g - 1, r1)
            return carry

        for i in range(_NB):
            fire_ids(i, i)

        body(0, 0, ring_wait=False, ids_ahead=True, do_prev=False)
        lax.fori_loop(
            1, _NB,
            functools.partial(body, ring_wait=False, ids_ahead=True,
                              do_prev=True),
            0, unroll=False)
        lax.fori_loop(
            _NB, _NCHUNK - _NB,
            functools.partial(body, ring_wait=True, ids_ahead=True,
                              do_prev=True),
            0, unroll=False)
        lax.fori_loop(
            _NCHUNK - _NB, _NCHUNK,
            functools.partial(body, ring_wait=True, ids_ahead=False,
                              do_prev=True),
            0, unroll=False)

        rlast = (_NCHUNK - 1) % _NB
        wait_emb(rlast)
        fire_out(_NCHUNK - 1, rlast)
        for i in range(_NB):
            wait_out(i)

    return k


_k = _make_kernel()


def kernel(tokens, vocab_map, embed_table):
    b = tokens.shape[0]
    vmap_ext = jnp.concatenate(
        [vocab_map, jnp.array([0, 2], dtype=vocab_map.dtype)])
    bos = jnp.full((b, 1), _VOCAB, dtype=tokens.dtype)
    eos = jnp.full((b, 1), _VOCAB + 1, dtype=tokens.dtype)
    tok_ext = jnp.concatenate([bos, tokens, eos], axis=1).reshape(-1, _C)
    out = _k(tok_ext, vmap_ext, embed_table)
    return out.reshape(b, _S, _D)
